# baseline (speedup 1.0000x reference)
import functools

import jax
import jax.numpy as jnp
from jax import lax
from jax.experimental import pallas as pl
from jax.experimental.pallas import tpu as pltpu
from jax.experimental.pallas import tpu_sc as plsc

N_NODES = 100000
N_EDGES = 3200000
NC, NS, LANES = 2, 16, 16
CHUNK = 128
K_IDX = 4
N_ACC = 100096
ROWS_PER_SUB = N_ACC // NS
EDGE_ROWS = 25088
E_PAD = EDGE_ROWS * CHUNK

_MESH = plsc.VectorSubcoreMesh(
    core_axis_name="c", subcore_axis_name="s", num_cores=NC, num_subcores=NS)

_SC_PARAMS = pltpu.CompilerParams(use_tc_tiling_on_sc=False)


def _idx_bufs():
    return [pltpu.VMEM((K_IDX, 2, CHUNK), jnp.int32) for _ in range(2)]


def _zero_and_sync(zeros_hbm, acc, sid):
    sl = pl.ds(sid * ROWS_PER_SUB, ROWS_PER_SUB)
    pltpu.sync_copy(zeros_hbm.at[sl], acc.at[sl])
    plsc.subcore_barrier()


def _dump(acc, out_hbm, cid, sid):
    plsc.subcore_barrier()
    sl = pl.ds(sid * ROWS_PER_SUB, ROWS_PER_SUB)
    pltpu.sync_copy(acc.at[sl], out_hbm.at[cid].at[sl])


def _sc_degree(rc2d, zeros_acc, width):
    rows_per_worker = EDGE_ROWS // (NC * NS)
    n_macro = rows_per_worker // K_IDX
    n_pairs = (n_macro - 2) // 2

    @functools.partial(
        pl.kernel,
        out_type=jax.ShapeDtypeStruct((NC, N_ACC, width), jnp.float32),
        mesh=_MESH,
        scratch_types=_idx_bufs() + [
            pltpu.VMEM((CHUNK, width), jnp.float32),
            pltpu.VMEM_SHARED((N_ACC, width), jnp.float32),
        ] + [pltpu.SemaphoreType.DMA] * (K_IDX + 2),
        compiler_params=_SC_PARAMS,
    )
    def body(rc_hbm, ones_hbm, zeros_hbm, out_hbm, idxA, idxB, ones_v, acc,
             *sems):
        ssem = sems[:K_IDX]
        isemA, isemB = sems[K_IDX], sems[K_IDX + 1]
        cid = lax.axis_index("c")
        sid = lax.axis_index("s")
        pltpu.sync_copy(ones_hbm, ones_v)
        _zero_and_sync(zeros_hbm, acc, sid)
        base = (cid * NS + sid) * rows_per_worker

        def idx_slice(m):
            return rc_hbm.at[pl.ds(base + m * K_IDX, K_IDX)]

        def wait_idx(buf, sem):
            pltpu.make_async_copy(idx_slice(0), buf, sem).wait()

        def half(ibuf, fire_next):
            for j in range(K_IDX):
                pltpu.async_copy(ones_v, acc.at[ibuf.at[j, 1]], ssem[j],
                                 add=True)
            for j in range(K_IDX):
                pltpu.make_async_copy(
                    ones_v, acc.at[ibuf.at[j, 1]], ssem[j]).wait()
            if fire_next is not None:
                m_next, buf, sem = fire_next
                pltpu.async_copy(idx_slice(m_next), buf, sem)

        pltpu.sync_copy(idx_slice(0), idxA)
        pltpu.async_copy(idx_slice(1), idxB, isemB)

        @pl.loop(0, n_pairs)
        def _(k):
            m = 2 * k
            wait_idx(idxB, isemB)
            half(idxA, (m + 2, idxA, isemA))
            wait_idx(idxA, isemA)
            half(idxB, (m + 3, idxB, isemB))

        wait_idx(idxB, isemB)
        half(idxA, None)
        half(idxB, None)

        _dump(acc, out_hbm, cid, sid)

    return body(rc2d, jnp.ones((CHUNK, width), jnp.float32), zeros_acc)


def _sc_aggregate(table, rc2d, zeros_acc, feat_split, width):
    n_workers = NS if feat_split else NC * NS
    rows_per_worker = EDGE_ROWS // n_workers
    n_macro = rows_per_worker // K_IDX
    n_pairs = (n_macro - 2) // 2

    @functools.partial(
        pl.kernel,
        out_type=jax.ShapeDtypeStruct((NC, N_ACC, width), jnp.float32),
        mesh=_MESH,
        scratch_types=_idx_bufs()
        + [pltpu.VMEM((CHUNK, width), jnp.float32) for _ in range(2 * K_IDX)]
        + [pltpu.VMEM_SHARED((N_ACC, width), jnp.float32)]
        + [pltpu.SemaphoreType.DMA] * (2 * K_IDX + 2),
        compiler_params=_SC_PARAMS,
    )
    def body(table_hbm, rc_hbm, zeros_hbm, out_hbm, idxA, idxB, *rest):
        bufsA = rest[:K_IDX]
        bufsB = rest[K_IDX:2 * K_IDX]
        acc = rest[2 * K_IDX]
        sems = rest[2 * K_IDX + 1:]
        gsem = sems[:K_IDX]
        ssem = sems[K_IDX:2 * K_IDX]
        isemA, isemB = sems[2 * K_IDX], sems[2 * K_IDX + 1]
        cid = lax.axis_index("c")
        sid = lax.axis_index("s")
        _zero_and_sync(zeros_hbm, acc, sid)

        tbl = table_hbm.at[cid] if feat_split else table_hbm
        wid = sid if feat_split else cid * NS + sid
        base = wid * rows_per_worker

        def idx_slice(m):
            return rc_hbm.at[pl.ds(base + m * K_IDX, K_IDX)]

        def wait_idx(buf, sem):
            pltpu.make_async_copy(idx_slice(0), buf, sem).wait()

        def fire_gathers(ibuf, bufs):
            for j in range(K_IDX):
                pltpu.async_copy(tbl.at[ibuf.at[j, 0]], bufs[j], gsem[j])

        def half(ibuf, bufs, nxt_ibuf, nxt_bufs, fire_next):
            for j in range(K_IDX):
                pltpu.make_async_copy(
                    tbl.at[ibuf.at[j, 0]], bufs[j], gsem[j]).wait()
                pltpu.async_copy(bufs[j], acc.at[ibuf.at[j, 1]], ssem[j],
                                 add=True)
                if nxt_ibuf is not None:
                    pltpu.async_copy(
                        tbl.at[nxt_ibuf.at[j, 0]], nxt_bufs[j], gsem[j])
            for j in range(K_IDX):
                pltpu.make_async_copy(
                    bufs[j], acc.at[ibuf.at[j, 1]], ssem[j]).wait()
            if fire_next is not None:
                m_next, buf, sem = fire_next
                pltpu.async_copy(idx_slice(m_next), buf, sem)

        pltpu.sync_copy(idx_slice(0), idxA)
        fire_gathers(idxA, bufsA)
        pltpu.async_copy(idx_slice(1), idxB, isemB)

        @pl.loop(0, n_pairs)
        def _(k):
            m = 2 * k
            wait_idx(idxB, isemB)
            half(idxA, bufsA, idxB, bufsB, (m + 2, idxA, isemA))
            wait_idx(idxA, isemA)
            half(idxB, bufsB, idxA, bufsA, (m + 3, idxB, isemB))

        wait_idx(idxB, isemB)
        half(idxA, bufsA, idxB, bufsB, None)
        half(idxB, bufsB, None, None, None)

        _dump(acc, out_hbm, cid, sid)

    return body(table, rc2d, zeros_acc)


R = 2000
GRID = N_NODES // R

_seq = pltpu.CompilerParams(dimension_semantics=("arbitrary",))


def _blk_nodes(*trail):
    return pl.BlockSpec((R,) + trail, lambda i: (i,) + (0,) * len(trail))


def _blk_acc(w=LANES):
    return pl.BlockSpec((NC, R, w), lambda i: (0, i, 0))


def _blk_full(shape):
    return pl.BlockSpec(shape, lambda i: (0,) * len(shape))


def _tck1_prep(acc_deg, x):
    def body(deg_ref, x_ref, dinv_ref, u1_ref):
        deg = deg_ref[0, :, 0] + deg_ref[1, :, 0] + 1.0
        dinv = (1.0 / jnp.sqrt(deg))[:, None]
        dinv_ref[...] = dinv
        u1_ref[...] = jnp.concatenate(
            [x_ref[...].astype(jnp.bfloat16).astype(jnp.float32) * dinv,
             jnp.zeros((R, LANES - 3), jnp.float32)], axis=1)

    return pl.pallas_call(
        body,
        grid=(GRID,),
        in_specs=[_blk_acc(), _blk_nodes(3)],
        out_specs=[_blk_nodes(1), _blk_nodes(LANES)],
        out_shape=[jax.ShapeDtypeStruct((N_NODES, 1), jnp.float32),
                   jax.ShapeDtypeStruct((N_NODES, LANES), jnp.float32)],
        compiler_params=_seq,
    )(acc_deg, x)


def _tck2_gcn1(acc1, u1, dinv, W1p, b1):
    def body(acc_ref, u1_ref, dinv_ref, w_ref, b_ref, g1_ref, sum_ref, sq_ref):
        s1 = (acc_ref[0] + acc_ref[1] + u1_ref[...]) * dinv_ref[...]
        g1 = jnp.dot(s1, w_ref[...], preferred_element_type=jnp.float32,
                     precision=lax.Precision.HIGHEST) + b_ref[...]
        g1_ref[...] = g1

        @pl.when(pl.program_id(0) == 0)
        def _():
            sum_ref[...] = jnp.zeros_like(sum_ref)
            sq_ref[...] = jnp.zeros_like(sq_ref)

        sum_ref[...] += jnp.sum(g1, axis=0, keepdims=True)
        sq_ref[...] += jnp.sum(g1 * g1, axis=0, keepdims=True)

    return pl.pallas_call(
        body,
        grid=(GRID,),
        in_specs=[_blk_acc(), _blk_nodes(LANES), _blk_nodes(1),
                  _blk_full((LANES, 64)), _blk_full((1, 64))],
        out_specs=[_blk_nodes(64), _blk_full((1, 64)), _blk_full((1, 64))],
        out_shape=[jax.ShapeDtypeStruct((N_NODES, 64), jnp.float32),
                   jax.ShapeDtypeStruct((1, 64), jnp.float32),
                   jax.ShapeDtypeStruct((1, 64), jnp.float32)],
        compiler_params=_seq,
    )(acc1, u1, dinv, W1p, b1)


def _tck3_bn1(g1, sums, sqs, gamma, beta, x, W_res1, b_res1, W_enc2, dinv):
    def body(g1_ref, sum_ref, sq_ref, ga_ref, be_ref, x_ref, wr_ref, br_ref,
             w2_ref, dinv_ref, h1_ref, u2_ref):
        mu = sum_ref[...] / N_NODES
        var = sq_ref[...] / N_NODES - mu * mu
        xn = (g1_ref[...] - mu) * lax.rsqrt(var + 1e-5) * ga_ref[...] + be_ref[...]
        res = jnp.dot(x_ref[...], wr_ref[...],
                      preferred_element_type=jnp.float32) + br_ref[...]
        h1 = jnp.maximum(xn, 0.0) + res
        h1_ref[...] = h1
        h2p = jnp.dot(h1, w2_ref[...], preferred_element_type=jnp.float32)
        u2 = h2p * dinv_ref[...]
        u2_ref[0] = u2[:, :LANES]
        u2_ref[1] = u2[:, LANES:]

    return pl.pallas_call(
        body,
        grid=(GRID,),
        in_specs=[_blk_nodes(64), _blk_full((1, 64)), _blk_full((1, 64)),
                  _blk_full((1, 64)), _blk_full((1, 64)), _blk_nodes(3),
                  _blk_full((3, 64)), _blk_full((1, 64)), _blk_full((64, 32)),
                  _blk_nodes(1)],
        out_specs=[_blk_nodes(64), _blk_acc()],
        out_shape=[jax.ShapeDtypeStruct((N_NODES, 64), jnp.float32),
                   jax.ShapeDtypeStruct((NC, N_NODES, LANES), jnp.float32)],
        compiler_params=_seq,
    )(g1, sums, sqs, gamma, beta, x, W_res1, b_res1, W_enc2, dinv)


def _tck4_gcn2(acc2, u2, dinv, b2):
    def body(acc_ref, u2_ref, dinv_ref, b_ref, g2_ref, sum_ref, sq_ref):
        lo = acc_ref[0] + u2_ref[0]
        hi = acc_ref[1] + u2_ref[1]
        g2 = jnp.concatenate([lo, hi], axis=1) * dinv_ref[...] + b_ref[...]
        g2_ref[...] = g2

        @pl.when(pl.program_id(0) == 0)
        def _():
            sum_ref[...] = jnp.zeros_like(sum_ref)
            sq_ref[...] = jnp.zeros_like(sq_ref)

        sum_ref[...] += jnp.sum(g2, axis=0, keepdims=True)
        sq_ref[...] += jnp.sum(g2 * g2, axis=0, keepdims=True)

    return pl.pallas_call(
        body,
        grid=(GRID,),
        in_specs=[_blk_acc(), pl.BlockSpec((NC, R, LANES), lambda i: (0, i, 0)),
                  _blk_nodes(1), _blk_full((1, 32))],
        out_specs=[_blk_nodes(32), _blk_full((1, 32)), _blk_full((1, 32))],
        out_shape=[jax.ShapeDtypeStruct((N_NODES, 32), jnp.float32),
                   jax.ShapeDtypeStruct((1, 32), jnp.float32),
                   jax.ShapeDtypeStruct((1, 32), jnp.float32)],
        compiler_params=_seq,
    )(acc2, u2, dinv, b2)


def _tck5_bn2(g2, sums, sqs, gamma, beta, h1, W_res2, b_res2, W_enc3, dinv):
    def body(g2_ref, sum_ref, sq_ref, ga_ref, be_ref, h1_ref, wr_ref, br_ref,
             w3_ref, dinv_ref, u3_ref):
        mu = sum_ref[...] / N_NODES
        var = sq_ref[...] / N_NODES - mu * mu
        xn = (g2_ref[...] - mu) * lax.rsqrt(var + 1e-5) * ga_ref[...] + be_ref[...]
        res = jnp.dot(h1_ref[...], wr_ref[...],
                      preferred_element_type=jnp.float32) + br_ref[...]
        h2 = jnp.maximum(xn, 0.0) + res
        h3p = jnp.dot(h2, w3_ref[...], preferred_element_type=jnp.float32)
        u3_ref[...] = jnp.concatenate(
            [h3p * dinv_ref[...], jnp.zeros((R, LANES - 1), jnp.float32)],
            axis=1)

    return pl.pallas_call(
        body,
        grid=(GRID,),
        in_specs=[_blk_nodes(32), _blk_full((1, 32)), _blk_full((1, 32)),
                  _blk_full((1, 32)), _blk_full((1, 32)), _blk_nodes(64),
                  _blk_full((64, 32)), _blk_full((1, 32)), _blk_full((32, 1)),
                  _blk_nodes(1)],
        out_specs=[_blk_nodes(LANES)],
        out_shape=[jax.ShapeDtypeStruct((N_NODES, LANES), jnp.float32)],
        compiler_params=_seq,
    )(g2, sums, sqs, gamma, beta, h1, W_res2, b_res2, W_enc3, dinv)[0]


def _tck6_heads(acc3, u3, dinv, b_enc3, W_dec1, b_dec1, W_dec2, b_dec2,
                W_dec3, b_dec3, W_t1, b_t1, W_t2, b_t2):
    def body(acc_ref, u3_ref, dinv_ref, be3_ref, wd1_ref, bd1_ref, wd2_ref,
             bd2_ref, wd3_ref, bd3_ref, wt1_ref, bt1_ref, wt2_ref, bt2_ref,
             recon_ref, t_ref, z_ref):
        z = ((acc_ref[0, :, 0:1] + acc_ref[1, :, 0:1] + u3_ref[:, 0:1])
             * dinv_ref[...]) + be3_ref[...]
        z_ref[...] = z
        d = jnp.maximum(jnp.dot(z, wd1_ref[...],
                                preferred_element_type=jnp.float32)
                        + bd1_ref[...], 0.0)
        d = jnp.maximum(jnp.dot(d, wd2_ref[...],
                                preferred_element_type=jnp.float32)
                        + bd2_ref[...], 0.0)
        recon_ref[...] = jnp.dot(d, wd3_ref[...],
                                 preferred_element_type=jnp.float32) + bd3_ref[...]
        t = jnp.maximum(jnp.dot(z, wt1_ref[...],
                                preferred_element_type=jnp.float32)
                        + bt1_ref[...], 0.0)
        t_ref[...] = jnp.dot(t, wt2_ref[...],
                             preferred_element_type=jnp.float32) + bt2_ref[...]

    return pl.pallas_call(
        body,
        grid=(GRID,),
        in_specs=[_blk_acc(), _blk_nodes(LANES), _blk_nodes(1),
                  _blk_full((1, 1)), _blk_full((1, 32)), _blk_full((1, 32)),
                  _blk_full((32, 64)), _blk_full((1, 64)), _blk_full((64, 3)),
                  _blk_full((1, 3)), _blk_full((1, 16)), _blk_full((1, 16)),
                  _blk_full((16, 1)), _blk_full((1, 1))],
        out_specs=[_blk_nodes(3), _blk_nodes(1), _blk_nodes(1)],
        out_shape=[jax.ShapeDtypeStruct((N_NODES, 3), jnp.float32),
                   jax.ShapeDtypeStruct((N_NODES, 1), jnp.float32),
                   jax.ShapeDtypeStruct((N_NODES, 1), jnp.float32)],
        compiler_params=_seq,
    )(acc3, u3, dinv, b_enc3, W_dec1, b_dec1, W_dec2, b_dec2, W_dec3, b_dec3,
      W_t1, b_t1, W_t2, b_t2)


def kernel(x, edge_index, W_enc1, b_enc1, W_res1, b_res1, gamma1, beta1,
           W_enc2, b_enc2, W_res2, b_res2, gamma2, beta2, W_enc3, b_enc3,
           W_dec1, b_dec1, W_dec2, b_dec2, W_dec3, b_dec3, W_t1, b_t1,
           W_t2, b_t2):
    pad = E_PAD - N_EDGES
    row = jnp.concatenate([edge_index[0], jnp.zeros((pad,), jnp.int32)])
    col = jnp.concatenate(
        [edge_index[1],
         N_NODES + (jnp.arange(pad, dtype=jnp.int32) % 64)])
    rc2d = jnp.stack(
        [row.reshape(EDGE_ROWS, CHUNK), col.reshape(EDGE_ROWS, CHUNK)], axis=1)
    zeros16 = jnp.zeros((N_ACC, LANES), jnp.float32)

    acc_deg = _sc_degree(rc2d, zeros16, LANES)
    dinv, u1 = _tck1_prep(acc_deg, x)

    acc1 = _sc_aggregate(u1, rc2d, zeros16, feat_split=False, width=LANES)
    W1b = W_enc1.astype(jnp.bfloat16).astype(jnp.float32)
    W1p = jnp.zeros((LANES, 64), jnp.float32).at[:3].set(W1b)
    g1, s1, q1 = _tck2_gcn1(acc1, u1, dinv, W1p, b_enc1.reshape(1, 64))
    h1, u2 = _tck3_bn1(g1, s1, q1, gamma1.reshape(1, 64), beta1.reshape(1, 64),
                       x, W_res1, b_res1.reshape(1, 64), W_enc2, dinv)

    acc2 = _sc_aggregate(u2, rc2d, zeros16, feat_split=True, width=LANES)
    g2, s2, q2 = _tck4_gcn2(acc2, u2, dinv, b_enc2.reshape(1, 32))
    u3 = _tck5_bn2(g2, s2, q2, gamma2.reshape(1, 32), beta2.reshape(1, 32),
                   h1, W_res2, b_res2.reshape(1, 32), W_enc3, dinv)

    acc3 = _sc_aggregate(u3, rc2d, zeros16, feat_split=False, width=LANES)
    recon, t, z = _tck6_heads(
        acc3, u3, dinv, b_enc3.reshape(1, 1), W_dec1, b_dec1.reshape(1, 32),
        W_dec2, b_dec2.reshape(1, 64), W_dec3, b_dec3.reshape(1, 3),
        W_t1, b_t1.reshape(1, 16), W_t2, b_t2.reshape(1, 1))
    return (recon, t[:, 0], z[:, 0])


# speedup vs baseline: 1.0208x; 1.0208x over previous
import functools

import jax
import jax.numpy as jnp
from jax import lax
from jax.experimental import pallas as pl
from jax.experimental.pallas import tpu as pltpu
from jax.experimental.pallas import tpu_sc as plsc

N_NODES = 100000
N_EDGES = 3200000
NC, NS, LANES = 2, 16, 16
CHUNK = 128
K_IDX = 4
N_ACC = 100096
ROWS_PER_SUB = N_ACC // NS
EDGE_ROWS = 25088
E_PAD = EDGE_ROWS * CHUNK

_MESH = plsc.VectorSubcoreMesh(
    core_axis_name="c", subcore_axis_name="s", num_cores=NC, num_subcores=NS)

_SC_PARAMS = pltpu.CompilerParams(use_tc_tiling_on_sc=False)


def _idx_bufs(n):
    return [pltpu.VMEM((K_IDX, CHUNK), jnp.int32) for _ in range(n)]


def _zero_and_sync(zeros_hbm, acc, sid):
    sl = pl.ds(sid * ROWS_PER_SUB, ROWS_PER_SUB)
    pltpu.sync_copy(zeros_hbm.at[sl], acc.at[sl])
    plsc.subcore_barrier()


def _dump(acc, out_hbm, cid, sid):
    plsc.subcore_barrier()
    sl = pl.ds(sid * ROWS_PER_SUB, ROWS_PER_SUB)
    pltpu.sync_copy(acc.at[sl], out_hbm.at[cid].at[sl])


def _sc_degree(col2d, zeros_acc, width):
    rows_per_worker = EDGE_ROWS // (NC * NS)
    n_macro = rows_per_worker // K_IDX
    n_pairs = (n_macro - 2) // 2

    @functools.partial(
        pl.kernel,
        out_type=jax.ShapeDtypeStruct((NC, N_ACC, width), jnp.float32),
        mesh=_MESH,
        scratch_types=_idx_bufs(2) + [
            pltpu.VMEM((CHUNK, width), jnp.float32),
            pltpu.VMEM_SHARED((N_ACC, width), jnp.float32),
        ] + [pltpu.SemaphoreType.DMA] * (K_IDX + 2),
        compiler_params=_SC_PARAMS,
    )
    def body(col_hbm, ones_hbm, zeros_hbm, out_hbm, idxA, idxB, ones_v, acc,
             *sems):
        ssem = sems[:K_IDX]
        isemA, isemB = sems[K_IDX], sems[K_IDX + 1]
        cid = lax.axis_index("c")
        sid = lax.axis_index("s")
        pltpu.sync_copy(ones_hbm, ones_v)
        _zero_and_sync(zeros_hbm, acc, sid)
        base = (cid * NS + sid) * rows_per_worker

        def idx_slice(m):
            return col_hbm.at[pl.ds(base + m * K_IDX, K_IDX)]

        def wait_idx(buf, sem):
            pltpu.make_async_copy(idx_slice(0), buf, sem).wait()

        def half(ibuf, fire_next):
            for j in range(K_IDX):
                pltpu.async_copy(ones_v, acc.at[ibuf.at[j]], ssem[j],
                                 add=True)
            for j in range(K_IDX):
                pltpu.make_async_copy(
                    ones_v, acc.at[ibuf.at[j]], ssem[j]).wait()
            if fire_next is not None:
                m_next, buf, sem = fire_next
                pltpu.async_copy(idx_slice(m_next), buf, sem)

        pltpu.sync_copy(idx_slice(0), idxA)
        pltpu.async_copy(idx_slice(1), idxB, isemB)

        @pl.loop(0, n_pairs)
        def _(k):
            m = 2 * k
            wait_idx(idxB, isemB)
            half(idxA, (m + 2, idxA, isemA))
            wait_idx(idxA, isemA)
            half(idxB, (m + 3, idxB, isemB))

        wait_idx(idxB, isemB)
        half(idxA, None)
        half(idxB, None)

        _dump(acc, out_hbm, cid, sid)

    return body(col2d, jnp.ones((CHUNK, width), jnp.float32), zeros_acc)


def _sc_aggregate(table, row2d, col2d, zeros_acc, feat_split, width):
    n_workers = NS if feat_split else NC * NS
    rows_per_worker = EDGE_ROWS // n_workers
    n_macro = rows_per_worker // K_IDX
    n_pairs = (n_macro - 2) // 2

    @functools.partial(
        pl.kernel,
        out_type=jax.ShapeDtypeStruct((NC, N_ACC, width), jnp.float32),
        mesh=_MESH,
        scratch_types=_idx_bufs(4)
        + [pltpu.VMEM((CHUNK, width), jnp.float32) for _ in range(2 * K_IDX)]
        + [pltpu.VMEM_SHARED((N_ACC, width), jnp.float32)]
        + [pltpu.SemaphoreType.DMA] * (2 * K_IDX + 2),
        compiler_params=_SC_PARAMS,
    )
    def body(table_hbm, row_hbm, col_hbm, zeros_hbm, out_hbm,
             rowA, colA, rowB, colB, *rest):
        bufsA = rest[:K_IDX]
        bufsB = rest[K_IDX:2 * K_IDX]
        acc = rest[2 * K_IDX]
        sems = rest[2 * K_IDX + 1:]
        gsem = sems[:K_IDX]
        ssem = sems[K_IDX:2 * K_IDX]
        isemA, isemB = sems[2 * K_IDX], sems[2 * K_IDX + 1]
        cid = lax.axis_index("c")
        sid = lax.axis_index("s")
        _zero_and_sync(zeros_hbm, acc, sid)

        tbl = table_hbm.at[cid] if feat_split else table_hbm
        wid = sid if feat_split else cid * NS + sid
        base = wid * rows_per_worker

        def row_slice(m):
            return row_hbm.at[pl.ds(base + m * K_IDX, K_IDX)]

        def col_slice(m):
            return col_hbm.at[pl.ds(base + m * K_IDX, K_IDX)]

        def fire_idx(m, rbuf, cbuf, sem):
            pltpu.async_copy(row_slice(m), rbuf, sem)
            pltpu.async_copy(col_slice(m), cbuf, sem)

        def wait_idx(rbuf, cbuf, sem):
            pltpu.make_async_copy(row_slice(0), rbuf, sem).wait()
            pltpu.make_async_copy(col_slice(0), cbuf, sem).wait()

        def fire_gathers(rbuf, bufs):
            for j in range(K_IDX):
                pltpu.async_copy(tbl.at[rbuf.at[j]], bufs[j], gsem[j])

        def half(rbuf, cbuf, bufs, nxt_rbuf, nxt_bufs, fire_next):
            for j in range(K_IDX):
                pltpu.make_async_copy(
                    tbl.at[rbuf.at[j]], bufs[j], gsem[j]).wait()
                pltpu.async_copy(bufs[j], acc.at[cbuf.at[j]], ssem[j],
                                 add=True)
                if nxt_rbuf is not None:
                    pltpu.async_copy(
                        tbl.at[nxt_rbuf.at[j]], nxt_bufs[j], gsem[j])
            for j in range(K_IDX):
                pltpu.make_async_copy(
                    bufs[j], acc.at[cbuf.at[j]], ssem[j]).wait()
            if fire_next is not None:
                m_next, rb, cb, sem = fire_next
                fire_idx(m_next, rb, cb, sem)

        pltpu.sync_copy(row_slice(0), rowA)
        pltpu.sync_copy(col_slice(0), colA)
        fire_gathers(rowA, bufsA)
        fire_idx(1, rowB, colB, isemB)

        @pl.loop(0, n_pairs)
        def _(k):
            m = 2 * k
            wait_idx(rowB, colB, isemB)
            half(rowA, colA, bufsA, rowB, bufsB, (m + 2, rowA, colA, isemA))
            wait_idx(rowA, colA, isemA)
            half(rowB, colB, bufsB, rowA, bufsA, (m + 3, rowB, colB, isemB))

        wait_idx(rowB, colB, isemB)
        half(rowA, colA, bufsA, rowB, bufsB, None)
        half(rowB, colB, bufsB, None, None, None)

        _dump(acc, out_hbm, cid, sid)

    return body(table, row2d, col2d, zeros_acc)


R = 2000
GRID = N_NODES // R

_seq = pltpu.CompilerParams(dimension_semantics=("arbitrary",))


def _blk_nodes(*trail):
    return pl.BlockSpec((R,) + trail, lambda i: (i,) + (0,) * len(trail))


def _blk_acc(w=LANES):
    return pl.BlockSpec((NC, R, w), lambda i: (0, i, 0))


def _blk_full(shape):
    return pl.BlockSpec(shape, lambda i: (0,) * len(shape))


def _tck1_prep(acc_deg, x):
    def body(deg_ref, x_ref, dinv_ref, u1_ref):
        deg = deg_ref[0, :, 0] + deg_ref[1, :, 0] + 1.0
        dinv = (1.0 / jnp.sqrt(deg))[:, None]
        dinv_ref[...] = dinv
        u1_ref[...] = jnp.concatenate(
            [x_ref[...].astype(jnp.bfloat16).astype(jnp.float32) * dinv,
             jnp.zeros((R, LANES - 3), jnp.float32)], axis=1)

    return pl.pallas_call(
        body,
        grid=(GRID,),
        in_specs=[_blk_acc(), _blk_nodes(3)],
        out_specs=[_blk_nodes(1), _blk_nodes(LANES)],
        out_shape=[jax.ShapeDtypeStruct((N_NODES, 1), jnp.float32),
                   jax.ShapeDtypeStruct((N_NODES, LANES), jnp.float32)],
        compiler_params=_seq,
    )(acc_deg, x)


def _tck2_gcn1(acc1, u1, dinv, W1p, b1):
    def body(acc_ref, u1_ref, dinv_ref, w_ref, b_ref, g1_ref, sum_ref, sq_ref):
        s1 = (acc_ref[0] + acc_ref[1] + u1_ref[...]) * dinv_ref[...]
        g1 = jnp.dot(s1, w_ref[...], preferred_element_type=jnp.float32,
                     precision=lax.Precision.HIGHEST) + b_ref[...]
        g1_ref[...] = g1

        @pl.when(pl.program_id(0) == 0)
        def _():
            sum_ref[...] = jnp.zeros_like(sum_ref)
            sq_ref[...] = jnp.zeros_like(sq_ref)

        sum_ref[...] += jnp.sum(g1, axis=0, keepdims=True)
        sq_ref[...] += jnp.sum(g1 * g1, axis=0, keepdims=True)

    return pl.pallas_call(
        body,
        grid=(GRID,),
        in_specs=[_blk_acc(), _blk_nodes(LANES), _blk_nodes(1),
                  _blk_full((LANES, 64)), _blk_full((1, 64))],
        out_specs=[_blk_nodes(64), _blk_full((1, 64)), _blk_full((1, 64))],
        out_shape=[jax.ShapeDtypeStruct((N_NODES, 64), jnp.float32),
                   jax.ShapeDtypeStruct((1, 64), jnp.float32),
                   jax.ShapeDtypeStruct((1, 64), jnp.float32)],
        compiler_params=_seq,
    )(acc1, u1, dinv, W1p, b1)


def _tck3_bn1(g1, sums, sqs, gamma, beta, x, W_res1, b_res1, W_enc2, dinv):
    def body(g1_ref, sum_ref, sq_ref, ga_ref, be_ref, x_ref, wr_ref, br_ref,
             w2_ref, dinv_ref, h1_ref, u2_ref):
        mu = sum_ref[...] / N_NODES
        var = sq_ref[...] / N_NODES - mu * mu
        xn = (g1_ref[...] - mu) * lax.rsqrt(var + 1e-5) * ga_ref[...] + be_ref[...]
        res = jnp.dot(x_ref[...], wr_ref[...],
                      preferred_element_type=jnp.float32) + br_ref[...]
        h1 = jnp.maximum(xn, 0.0) + res
        h1_ref[...] = h1
        h2p = jnp.dot(h1, w2_ref[...], preferred_element_type=jnp.float32)
        u2 = h2p * dinv_ref[...]
        u2_ref[0] = u2[:, :LANES]
        u2_ref[1] = u2[:, LANES:]

    return pl.pallas_call(
        body,
        grid=(GRID,),
        in_specs=[_blk_nodes(64), _blk_full((1, 64)), _blk_full((1, 64)),
                  _blk_full((1, 64)), _blk_full((1, 64)), _blk_nodes(3),
                  _blk_full((3, 64)), _blk_full((1, 64)), _blk_full((64, 32)),
                  _blk_nodes(1)],
        out_specs=[_blk_nodes(64), _blk_acc()],
        out_shape=[jax.ShapeDtypeStruct((N_NODES, 64), jnp.float32),
                   jax.ShapeDtypeStruct((NC, N_NODES, LANES), jnp.float32)],
        compiler_params=_seq,
    )(g1, sums, sqs, gamma, beta, x, W_res1, b_res1, W_enc2, dinv)


def _tck4_gcn2(acc2, u2, dinv, b2):
    def body(acc_ref, u2_ref, dinv_ref, b_ref, g2_ref, sum_ref, sq_ref):
        lo = acc_ref[0] + u2_ref[0]
        hi = acc_ref[1] + u2_ref[1]
        g2 = jnp.concatenate([lo, hi], axis=1) * dinv_ref[...] + b_ref[...]
        g2_ref[...] = g2

        @pl.when(pl.program_id(0) == 0)
        def _():
            sum_ref[...] = jnp.zeros_like(sum_ref)
            sq_ref[...] = jnp.zeros_like(sq_ref)

        sum_ref[...] += jnp.sum(g2, axis=0, keepdims=True)
        sq_ref[...] += jnp.sum(g2 * g2, axis=0, keepdims=True)

    return pl.pallas_call(
        body,
        grid=(GRID,),
        in_specs=[_blk_acc(), pl.BlockSpec((NC, R, LANES), lambda i: (0, i, 0)),
                  _blk_nodes(1), _blk_full((1, 32))],
        out_specs=[_blk_nodes(32), _blk_full((1, 32)), _blk_full((1, 32))],
        out_shape=[jax.ShapeDtypeStruct((N_NODES, 32), jnp.float32),
                   jax.ShapeDtypeStruct((1, 32), jnp.float32),
                   jax.ShapeDtypeStruct((1, 32), jnp.float32)],
        compiler_params=_seq,
    )(acc2, u2, dinv, b2)


def _tck5_bn2(g2, sums, sqs, gamma, beta, h1, W_res2, b_res2, W_enc3, dinv):
    def body(g2_ref, sum_ref, sq_ref, ga_ref, be_ref, h1_ref, wr_ref, br_ref,
             w3_ref, dinv_ref, u3_ref):
        mu = sum_ref[...] / N_NODES
        var = sq_ref[...] / N_NODES - mu * mu
        xn = (g2_ref[...] - mu) * lax.rsqrt(var + 1e-5) * ga_ref[...] + be_ref[...]
        res = jnp.dot(h1_ref[...], wr_ref[...],
                      preferred_element_type=jnp.float32) + br_ref[...]
        h2 = jnp.maximum(xn, 0.0) + res
        h3p = jnp.dot(h2, w3_ref[...], preferred_element_type=jnp.float32)
        u3_ref[...] = jnp.concatenate(
            [h3p * dinv_ref[...], jnp.zeros((R, LANES - 1), jnp.float32)],
            axis=1)

    return pl.pallas_call(
        body,
        grid=(GRID,),
        in_specs=[_blk_nodes(32), _blk_full((1, 32)), _blk_full((1, 32)),
                  _blk_full((1, 32)), _blk_full((1, 32)), _blk_nodes(64),
                  _blk_full((64, 32)), _blk_full((1, 32)), _blk_full((32, 1)),
                  _blk_nodes(1)],
        out_specs=[_blk_nodes(LANES)],
        out_shape=[jax.ShapeDtypeStruct((N_NODES, LANES), jnp.float32)],
        compiler_params=_seq,
    )(g2, sums, sqs, gamma, beta, h1, W_res2, b_res2, W_enc3, dinv)[0]


def _tck6_heads(acc3, u3, dinv, b_enc3, W_dec1, b_dec1, W_dec2, b_dec2,
                W_dec3, b_dec3, W_t1, b_t1, W_t2, b_t2):
    def body(acc_ref, u3_ref, dinv_ref, be3_ref, wd1_ref, bd1_ref, wd2_ref,
             bd2_ref, wd3_ref, bd3_ref, wt1_ref, bt1_ref, wt2_ref, bt2_ref,
             recon_ref, t_ref, z_ref):
        z = ((acc_ref[0, :, 0:1] + acc_ref[1, :, 0:1] + u3_ref[:, 0:1])
             * dinv_ref[...]) + be3_ref[...]
        z_ref[...] = z
        d = jnp.maximum(jnp.dot(z, wd1_ref[...],
                                preferred_element_type=jnp.float32)
                        + bd1_ref[...], 0.0)
        d = jnp.maximum(jnp.dot(d, wd2_ref[...],
                                preferred_element_type=jnp.float32)
                        + bd2_ref[...], 0.0)
        recon_ref[...] = jnp.dot(d, wd3_ref[...],
                                 preferred_element_type=jnp.float32) + bd3_ref[...]
        t = jnp.maximum(jnp.dot(z, wt1_ref[...],
                                preferred_element_type=jnp.float32)
                        + bt1_ref[...], 0.0)
        t_ref[...] = jnp.dot(t, wt2_ref[...],
                             preferred_element_type=jnp.float32) + bt2_ref[...]

    return pl.pallas_call(
        body,
        grid=(GRID,),
        in_specs=[_blk_acc(), _blk_nodes(LANES), _blk_nodes(1),
                  _blk_full((1, 1)), _blk_full((1, 32)), _blk_full((1, 32)),
                  _blk_full((32, 64)), _blk_full((1, 64)), _blk_full((64, 3)),
                  _blk_full((1, 3)), _blk_full((1, 16)), _blk_full((1, 16)),
                  _blk_full((16, 1)), _blk_full((1, 1))],
        out_specs=[_blk_nodes(3), _blk_nodes(1), _blk_nodes(1)],
        out_shape=[jax.ShapeDtypeStruct((N_NODES, 3), jnp.float32),
                   jax.ShapeDtypeStruct((N_NODES, 1), jnp.float32),
                   jax.ShapeDtypeStruct((N_NODES, 1), jnp.float32)],
        compiler_params=_seq,
    )(acc3, u3, dinv, b_enc3, W_dec1, b_dec1, W_dec2, b_dec2, W_dec3, b_dec3,
      W_t1, b_t1, W_t2, b_t2)


def kernel(x, edge_index, W_enc1, b_enc1, W_res1, b_res1, gamma1, beta1,
           W_enc2, b_enc2, W_res2, b_res2, gamma2, beta2, W_enc3, b_enc3,
           W_dec1, b_dec1, W_dec2, b_dec2, W_dec3, b_dec3, W_t1, b_t1,
           W_t2, b_t2):
    pad = E_PAD - N_EDGES
    row = jnp.concatenate([edge_index[0], jnp.zeros((pad,), jnp.int32)])
    col = jnp.concatenate(
        [edge_index[1],
         N_NODES + (jnp.arange(pad, dtype=jnp.int32) % 64)])
    row2d = row.reshape(EDGE_ROWS, CHUNK)
    col2d = col.reshape(EDGE_ROWS, CHUNK)
    zeros16 = jnp.zeros((N_ACC, LANES), jnp.float32)

    acc_deg = _sc_degree(col2d, zeros16, LANES)
    dinv, u1 = _tck1_prep(acc_deg, x)

    acc1 = _sc_aggregate(u1, row2d, col2d, zeros16, feat_split=False, width=LANES)
    W1b = W_enc1.astype(jnp.bfloat16).astype(jnp.float32)
    W1p = jnp.zeros((LANES, 64), jnp.float32).at[:3].set(W1b)
    g1, s1, q1 = _tck2_gcn1(acc1, u1, dinv, W1p, b_enc1.reshape(1, 64))
    h1, u2 = _tck3_bn1(g1, s1, q1, gamma1.reshape(1, 64), beta1.reshape(1, 64),
                       x, W_res1, b_res1.reshape(1, 64), W_enc2, dinv)

    acc2 = _sc_aggregate(u2, row2d, col2d, zeros16, feat_split=True, width=LANES)
    g2, s2, q2 = _tck4_gcn2(acc2, u2, dinv, b_enc2.reshape(1, 32))
    u3 = _tck5_bn2(g2, s2, q2, gamma2.reshape(1, 32), beta2.reshape(1, 32),
                   h1, W_res2, b_res2.reshape(1, 32), W_enc3, dinv)

    acc3 = _sc_aggregate(u3, row2d, col2d, zeros16, feat_split=False, width=LANES)
    recon, t, z = _tck6_heads(
        acc3, u3, dinv, b_enc3.reshape(1, 1), W_dec1, b_dec1.reshape(1, 32),
        W_dec2, b_dec2.reshape(1, 64), W_dec3, b_dec3.reshape(1, 3),
        W_t1, b_t1.reshape(1, 16), W_t2, b_t2.reshape(1, 1))
    return (recon, t[:, 0], z[:, 0])
